# baseline (speedup 1.0000x reference)
import functools

import jax
import jax.numpy as jnp
from jax import lax
from jax.experimental import pallas as pl
from jax.experimental.pallas import tpu as pltpu
from jax.experimental.pallas import tpu_sc as plsc

NUM_FACTORS = 32
VOCAB = 1000001
BATCH = 16384
H0, H1, H2 = 64, 32, 16

NC, NS = 2, 16
NW = NC * NS
BPW = BATCH // NW
CH = 128
NCH = BPW // CH

CBLK = 8192
BLK = 2048


def _tc_transpose_body(in_ref, out_ref):
    r = lax.broadcasted_iota(jnp.int32, (NUM_FACTORS, NUM_FACTORS), 0)
    c = lax.broadcasted_iota(jnp.int32, (NUM_FACTORS, NUM_FACTORS), 1)
    eye = (r == c).astype(jnp.float32)
    out_ref[...] = lax.dot_general(
        in_ref[...], eye, (((0,), (0,)), ((), ())),
        precision=lax.Precision.HIGHEST)


def _tc_transpose(tbl_t):
    n_blk = pl.cdiv(VOCAB, CBLK)
    return pl.pallas_call(
        _tc_transpose_body,
        grid=(n_blk,),
        in_specs=[pl.BlockSpec((NUM_FACTORS, CBLK), lambda i: (0, i))],
        out_specs=pl.BlockSpec((CBLK, NUM_FACTORS), lambda i: (i, 0)),
        out_shape=jax.ShapeDtypeStruct((VOCAB, NUM_FACTORS), jnp.float32),
    )(tbl_t)


def _sc_gather(uid2d, iid2d, P, Q, U, V):
    mesh = plsc.VectorSubcoreMesh(core_axis_name="c", subcore_axis_name="s")
    out_t = tuple(jax.ShapeDtypeStruct((BATCH, NUM_FACTORS), jnp.float32)
                  for _ in range(4))

    @functools.partial(
        pl.kernel, mesh=mesh, out_type=out_t,
        compiler_params=pltpu.CompilerParams(use_tc_tiling_on_sc=False),
        scratch_types=[
            pltpu.VMEM((NCH, CH), jnp.int32),
            pltpu.VMEM((NCH, CH), jnp.int32),
            pltpu.VMEM((BPW, NUM_FACTORS), jnp.float32),
            pltpu.VMEM((BPW, NUM_FACTORS), jnp.float32),
            pltpu.VMEM((BPW, NUM_FACTORS), jnp.float32),
            pltpu.VMEM((BPW, NUM_FACTORS), jnp.float32),
            pltpu.SemaphoreType.DMA,
        ],
    )
    def gather_kernel(uid_hbm, iid_hbm, p_hbm, q_hbm, u_hbm, v_hbm,
                      p_out, q_out, u_out, v_out,
                      uidx, iidx, pr, qr, ur, vr, sem):
        wid = lax.axis_index("s") * NC + lax.axis_index("c")
        row0 = wid * NCH
        pltpu.sync_copy(uid_hbm.at[pl.ds(row0, NCH)], uidx)
        pltpu.sync_copy(iid_hbm.at[pl.ds(row0, NCH)], iidx)
        copies = []
        for j in range(NCH):
            sl = pl.ds(j * CH, CH)
            copies.append(pltpu.async_copy(p_hbm.at[uidx.at[j]], pr.at[sl], sem))
            copies.append(pltpu.async_copy(u_hbm.at[uidx.at[j]], ur.at[sl], sem))
            copies.append(pltpu.async_copy(q_hbm.at[iidx.at[j]], qr.at[sl], sem))
            copies.append(pltpu.async_copy(v_hbm.at[iidx.at[j]], vr.at[sl], sem))
        for c in copies:
            c.wait()
        base = wid * BPW
        pltpu.sync_copy(pr, p_out.at[pl.ds(base, BPW)])
        pltpu.sync_copy(qr, q_out.at[pl.ds(base, BPW)])
        pltpu.sync_copy(ur, u_out.at[pl.ds(base, BPW)])
        pltpu.sync_copy(vr, v_out.at[pl.ds(base, BPW)])

    return gather_kernel(uid2d, iid2d, P, Q, U, V)


def _mlp_body(p_ref, q_ref, u_ref, v_ref, w0_ref, b0_ref, w1_ref, b1_ref,
              w2_ref, b2_ref, wp_ref, out_ref):
    hi = lax.Precision.HIGHEST
    gmf = p_ref[...] * q_ref[...]
    w0 = w0_ref[...]
    h = (jnp.dot(u_ref[...], w0[:NUM_FACTORS], precision=hi)
         + jnp.dot(v_ref[...], w0[NUM_FACTORS:], precision=hi) + b0_ref[...])
    h = jnp.maximum(h, 0.0)
    h = jnp.maximum(jnp.dot(h, w1_ref[...], precision=hi) + b1_ref[...], 0.0)
    h = jnp.maximum(jnp.dot(h, w2_ref[...], precision=hi) + b2_ref[...], 0.0)
    wp = wp_ref[...]
    out_ref[...] = (jnp.dot(gmf, wp[:NUM_FACTORS], precision=hi)
                    + jnp.dot(h, wp[NUM_FACTORS:], precision=hi))


def _mlp(p, q, u, v, W0, b0, W1, b1, W2, b2, Wp):
    n_blk = BATCH // BLK
    row_spec = lambda d: pl.BlockSpec((BLK, d), lambda i: (i, 0))
    full = lambda s: pl.BlockSpec(s, lambda i: (0, 0))
    return pl.pallas_call(
        _mlp_body,
        grid=(n_blk,),
        in_specs=[
            row_spec(NUM_FACTORS), row_spec(NUM_FACTORS),
            row_spec(NUM_FACTORS), row_spec(NUM_FACTORS),
            full((2 * NUM_FACTORS, H0)), full((1, H0)),
            full((H0, H1)), full((1, H1)),
            full((H1, H2)), full((1, H2)),
            full((H2 + NUM_FACTORS, 1)),
        ],
        out_specs=pl.BlockSpec((BLK, 1), lambda i: (i, 0)),
        out_shape=jax.ShapeDtypeStruct((BATCH, 1), jnp.float32),
    )(p, q, u, v, W0, b0.reshape(1, H0), W1, b1.reshape(1, H1),
      W2, b2.reshape(1, H2), Wp)


def kernel(user_id, item_id, P, Q, U, V, W0, b0, W1, b1, W2, b2, Wp):
    uid2d = user_id.astype(jnp.int32).reshape(BATCH // CH, CH)
    iid2d = item_id.astype(jnp.int32).reshape(BATCH // CH, CH)
    Pr = _tc_transpose(P.T)
    Qr = _tc_transpose(Q.T)
    Ur = _tc_transpose(U.T)
    Vr = _tc_transpose(V.T)
    p, q, u, v = _sc_gather(uid2d, iid2d, Pr, Qr, Ur, Vr)
    return _mlp(p, q, u, v, W0, b0, W1, b1, W2, b2, Wp)


# speedup vs baseline: 2.8007x; 2.8007x over previous
import functools

import jax
import jax.numpy as jnp
from jax import lax
from jax.experimental import pallas as pl
from jax.experimental.pallas import tpu as pltpu
from jax.experimental.pallas import tpu_sc as plsc

NUM_FACTORS = 32
VOCAB = 1000001
BATCH = 16384
H0, H1, H2 = 64, 32, 16

NC, NS = 2, 16
NW = NC * NS
BPW = BATCH // NW
CH = 128
NCH = BPW // CH

CBLK = 8192
QB = CBLK // 4
NBLK = pl.cdiv(VOCAB, CBLK)
NPACK = NBLK * QB
BLK = 2048


def _repack_body(in_ref, out_ref):
    x = in_ref[...]
    qs = [x[:, c * QB:(c + 1) * QB].T for c in range(4)]
    out_ref[...] = jnp.concatenate(qs, axis=1)


def _tc_repack(tbl_t):
    return pl.pallas_call(
        _repack_body,
        grid=(NBLK,),
        in_specs=[pl.BlockSpec((NUM_FACTORS, CBLK), lambda i: (0, i))],
        out_specs=pl.BlockSpec((QB, 4 * NUM_FACTORS), lambda i: (i, 0)),
        out_shape=jax.ShapeDtypeStruct((NPACK, 4 * NUM_FACTORS), jnp.float32),
    )(tbl_t)


def _sc_gather(gu, gi, Pp, Qp, Up, Vp):
    mesh = plsc.VectorSubcoreMesh(core_axis_name="c", subcore_axis_name="s")
    out_t = tuple(jax.ShapeDtypeStruct((BATCH, 4 * NUM_FACTORS), jnp.float32)
                  for _ in range(4))

    @functools.partial(
        pl.kernel, mesh=mesh, out_type=out_t,
        scratch_types=[
            pltpu.VMEM((CH,), jnp.int32),
            pltpu.VMEM((CH,), jnp.int32),
            pltpu.VMEM((CH, 4 * NUM_FACTORS), jnp.float32),
            pltpu.VMEM((CH, 4 * NUM_FACTORS), jnp.float32),
            pltpu.VMEM((CH, 4 * NUM_FACTORS), jnp.float32),
            pltpu.VMEM((CH, 4 * NUM_FACTORS), jnp.float32),
            pltpu.SemaphoreType.DMA,
        ],
    )
    def gather_kernel(gu_hbm, gi_hbm, p_hbm, q_hbm, u_hbm, v_hbm,
                      p_out, q_out, u_out, v_out,
                      uidx, iidx, pr, qr, ur, vr, sem):
        wid = lax.axis_index("s") * NC + lax.axis_index("c")
        base = wid * BPW

        def chunk(j):
            off = base + j * CH
            pltpu.sync_copy(gu_hbm.at[pl.ds(off, CH)], uidx)
            pltpu.sync_copy(gi_hbm.at[pl.ds(off, CH)], iidx)
            copies = [
                pltpu.async_copy(p_hbm.at[uidx], pr, sem),
                pltpu.async_copy(u_hbm.at[uidx], ur, sem),
                pltpu.async_copy(q_hbm.at[iidx], qr, sem),
                pltpu.async_copy(v_hbm.at[iidx], vr, sem),
            ]
            for c in copies:
                c.wait()
            sl = pl.ds(off, CH)
            pltpu.sync_copy(pr, p_out.at[sl])
            pltpu.sync_copy(qr, q_out.at[sl])
            pltpu.sync_copy(ur, u_out.at[sl])
            pltpu.sync_copy(vr, v_out.at[sl])

        for j in range(NCH):
            chunk(j)

    return gather_kernel(gu, gi, Pp, Qp, Up, Vp)


def _extract(g, off_ref):
    out = jnp.zeros((g.shape[0], NUM_FACTORS), jnp.float32)
    for c in range(4):
        sel = off_ref == (c * NUM_FACTORS)
        out = jnp.where(sel, g[:, c * NUM_FACTORS:(c + 1) * NUM_FACTORS], out)
    return out


def _mlp_body(p_ref, q_ref, u_ref, v_ref, ou_ref, oi_ref,
              w0_ref, b0_ref, w1_ref, b1_ref, w2_ref, b2_ref, wp_ref,
              out_ref):
    hi = lax.Precision.HIGHEST
    ou = ou_ref[...]
    oi = oi_ref[...]
    p = _extract(p_ref[...], ou)
    q = _extract(q_ref[...], oi)
    u = _extract(u_ref[...], ou)
    v = _extract(v_ref[...], oi)
    gmf = p * q
    w0 = w0_ref[...]
    h = (jnp.dot(u, w0[:NUM_FACTORS], precision=hi)
         + jnp.dot(v, w0[NUM_FACTORS:], precision=hi) + b0_ref[...])
    h = jnp.maximum(h, 0.0)
    h = jnp.maximum(jnp.dot(h, w1_ref[...], precision=hi) + b1_ref[...], 0.0)
    h = jnp.maximum(jnp.dot(h, w2_ref[...], precision=hi) + b2_ref[...], 0.0)
    wp = wp_ref[...]
    out_ref[...] = (jnp.dot(gmf, wp[:NUM_FACTORS], precision=hi)
                    + jnp.dot(h, wp[NUM_FACTORS:], precision=hi))


def _mlp(gp, gq, gu, gv, ou, oi, W0, b0, W1, b1, W2, b2, Wp):
    n_blk = BATCH // BLK
    row_spec = lambda d: pl.BlockSpec((BLK, d), lambda i: (i, 0))
    full = lambda s: pl.BlockSpec(s, lambda i: (0, 0))
    return pl.pallas_call(
        _mlp_body,
        grid=(n_blk,),
        in_specs=[
            row_spec(4 * NUM_FACTORS), row_spec(4 * NUM_FACTORS),
            row_spec(4 * NUM_FACTORS), row_spec(4 * NUM_FACTORS),
            row_spec(1), row_spec(1),
            full((2 * NUM_FACTORS, H0)), full((1, H0)),
            full((H0, H1)), full((1, H1)),
            full((H1, H2)), full((1, H2)),
            full((H2 + NUM_FACTORS, 1)),
        ],
        out_specs=pl.BlockSpec((BLK, 1), lambda i: (i, 0)),
        out_shape=jax.ShapeDtypeStruct((BATCH, 1), jnp.float32),
    )(gp, gq, gu, gv, ou, oi, W0, b0.reshape(1, H0), W1, b1.reshape(1, H1),
      W2, b2.reshape(1, H2), Wp)


def kernel(user_id, item_id, P, Q, U, V, W0, b0, W1, b1, W2, b2, Wp):
    uid = user_id.astype(jnp.int32)
    iid = item_id.astype(jnp.int32)
    last_full = (NBLK - 1) * CBLK
    clamp_start = VOCAB - CBLK

    def packed_coords(idx):
        tail = idx >= last_full
        b = jnp.where(tail, NBLK - 1, idx // CBLK)
        r = jnp.where(tail, idx - clamp_start, idx % CBLK)
        g = b * QB + r % QB
        off = (r // QB) * NUM_FACTORS
        return g, off

    gu, ou = packed_coords(uid)
    gi, oi = packed_coords(iid)
    Pp = _tc_repack(P.T)
    Qp = _tc_repack(Q.T)
    Up = _tc_repack(U.T)
    Vp = _tc_repack(V.T)
    gp, gq, gub, gvb = _sc_gather(gu, gi, Pp, Qp, Up, Vp)
    return _mlp(gp, gq, gub, gvb, ou.reshape(BATCH, 1), oi.reshape(BATCH, 1),
                W0, b0, W1, b1, W2, b2, Wp)


# speedup vs baseline: 2.8452x; 1.0159x over previous
import functools

import jax
import jax.numpy as jnp
from jax import lax
from jax.experimental import pallas as pl
from jax.experimental.pallas import tpu as pltpu
from jax.experimental.pallas import tpu_sc as plsc

NUM_FACTORS = 32
VOCAB = 1000001
BATCH = 16384
H0, H1, H2 = 64, 32, 16

NC, NS = 2, 16
NW = NC * NS
BPW = BATCH // NW
CH = 128
NCH = BPW // CH

CBLK = 16384
QB = CBLK // 4
NBLK = pl.cdiv(VOCAB, CBLK)
NPACK = NBLK * QB
BLK = 2048


def _repack_body(in_ref, out_ref):
    x = in_ref[...]
    qs = [x[:, c * QB:(c + 1) * QB].T for c in range(4)]
    out_ref[...] = jnp.concatenate(qs, axis=1)


def _tc_repack(tbl_t):
    return pl.pallas_call(
        _repack_body,
        grid=(NBLK,),
        in_specs=[pl.BlockSpec((NUM_FACTORS, CBLK), lambda i: (0, i))],
        out_specs=pl.BlockSpec((QB, 4 * NUM_FACTORS), lambda i: (i, 0)),
        out_shape=jax.ShapeDtypeStruct((NPACK, 4 * NUM_FACTORS), jnp.float32),
    )(tbl_t)


def _sc_gather(gu, gi, Pp, Qp, Up, Vp):
    mesh = plsc.VectorSubcoreMesh(core_axis_name="c", subcore_axis_name="s")
    out_t = tuple(jax.ShapeDtypeStruct((BATCH, 4 * NUM_FACTORS), jnp.float32)
                  for _ in range(4))

    @functools.partial(
        pl.kernel, mesh=mesh, out_type=out_t,
        scratch_types=[
            pltpu.VMEM((CH,), jnp.int32),
            pltpu.VMEM((CH,), jnp.int32),
            pltpu.VMEM((CH, 4 * NUM_FACTORS), jnp.float32),
            pltpu.VMEM((CH, 4 * NUM_FACTORS), jnp.float32),
            pltpu.VMEM((CH, 4 * NUM_FACTORS), jnp.float32),
            pltpu.VMEM((CH, 4 * NUM_FACTORS), jnp.float32),
            pltpu.SemaphoreType.DMA,
        ],
    )
    def gather_kernel(gu_hbm, gi_hbm, p_hbm, q_hbm, u_hbm, v_hbm,
                      p_out, q_out, u_out, v_out,
                      uidx, iidx, pr, qr, ur, vr, sem):
        wid = lax.axis_index("s") * NC + lax.axis_index("c")
        base = wid * BPW

        def chunk(j):
            off = base + j * CH
            pltpu.sync_copy(gu_hbm.at[pl.ds(off, CH)], uidx)
            pltpu.sync_copy(gi_hbm.at[pl.ds(off, CH)], iidx)
            copies = [
                pltpu.async_copy(p_hbm.at[uidx], pr, sem),
                pltpu.async_copy(u_hbm.at[uidx], ur, sem),
                pltpu.async_copy(q_hbm.at[iidx], qr, sem),
                pltpu.async_copy(v_hbm.at[iidx], vr, sem),
            ]
            for c in copies:
                c.wait()
            sl = pl.ds(off, CH)
            pltpu.sync_copy(pr, p_out.at[sl])
            pltpu.sync_copy(qr, q_out.at[sl])
            pltpu.sync_copy(ur, u_out.at[sl])
            pltpu.sync_copy(vr, v_out.at[sl])

        for j in range(NCH):
            chunk(j)

    return gather_kernel(gu, gi, Pp, Qp, Up, Vp)


def _extract(g, off_ref):
    out = jnp.zeros((g.shape[0], NUM_FACTORS), jnp.float32)
    for c in range(4):
        sel = off_ref == (c * NUM_FACTORS)
        out = jnp.where(sel, g[:, c * NUM_FACTORS:(c + 1) * NUM_FACTORS], out)
    return out


def _mlp_body(p_ref, q_ref, u_ref, v_ref, ou_ref, oi_ref,
              w0_ref, b0_ref, w1_ref, b1_ref, w2_ref, b2_ref, wp_ref,
              out_ref):
    hi = lax.Precision.HIGHEST
    ou = ou_ref[...]
    oi = oi_ref[...]
    p = _extract(p_ref[...], ou)
    q = _extract(q_ref[...], oi)
    u = _extract(u_ref[...], ou)
    v = _extract(v_ref[...], oi)
    gmf = p * q
    w0 = w0_ref[...]
    h = (jnp.dot(u, w0[:NUM_FACTORS], precision=hi)
         + jnp.dot(v, w0[NUM_FACTORS:], precision=hi) + b0_ref[...])
    h = jnp.maximum(h, 0.0)
    h = jnp.maximum(jnp.dot(h, w1_ref[...], precision=hi) + b1_ref[...], 0.0)
    h = jnp.maximum(jnp.dot(h, w2_ref[...], precision=hi) + b2_ref[...], 0.0)
    wp = wp_ref[...]
    out_ref[...] = (jnp.dot(gmf, wp[:NUM_FACTORS], precision=hi)
                    + jnp.dot(h, wp[NUM_FACTORS:], precision=hi))


def _mlp(gp, gq, gu, gv, ou, oi, W0, b0, W1, b1, W2, b2, Wp):
    n_blk = BATCH // BLK
    row_spec = lambda d: pl.BlockSpec((BLK, d), lambda i: (i, 0))
    full = lambda s: pl.BlockSpec(s, lambda i: (0, 0))
    return pl.pallas_call(
        _mlp_body,
        grid=(n_blk,),
        in_specs=[
            row_spec(4 * NUM_FACTORS), row_spec(4 * NUM_FACTORS),
            row_spec(4 * NUM_FACTORS), row_spec(4 * NUM_FACTORS),
            row_spec(1), row_spec(1),
            full((2 * NUM_FACTORS, H0)), full((1, H0)),
            full((H0, H1)), full((1, H1)),
            full((H1, H2)), full((1, H2)),
            full((H2 + NUM_FACTORS, 1)),
        ],
        out_specs=pl.BlockSpec((BLK, 1), lambda i: (i, 0)),
        out_shape=jax.ShapeDtypeStruct((BATCH, 1), jnp.float32),
    )(gp, gq, gu, gv, ou, oi, W0, b0.reshape(1, H0), W1, b1.reshape(1, H1),
      W2, b2.reshape(1, H2), Wp)


def kernel(user_id, item_id, P, Q, U, V, W0, b0, W1, b1, W2, b2, Wp):
    uid = user_id.astype(jnp.int32)
    iid = item_id.astype(jnp.int32)
    last_full = (NBLK - 1) * CBLK
    clamp_start = VOCAB - CBLK

    def packed_coords(idx):
        tail = idx >= last_full
        b = jnp.where(tail, NBLK - 1, idx // CBLK)
        r = jnp.where(tail, idx - clamp_start, idx % CBLK)
        g = b * QB + r % QB
        off = (r // QB) * NUM_FACTORS
        return g, off

    gu, ou = packed_coords(uid)
    gi, oi = packed_coords(iid)
    Pp = _tc_repack(P.T)
    Qp = _tc_repack(Q.T)
    Up = _tc_repack(U.T)
    Vp = _tc_repack(V.T)
    gp, gq, gub, gvb = _sc_gather(gu, gi, Pp, Qp, Up, Vp)
    return _mlp(gp, gq, gub, gvb, ou.reshape(BATCH, 1), oi.reshape(BATCH, 1),
                W0, b0, W1, b1, W2, b2, Wp)


# speedup vs baseline: 5.8102x; 2.0421x over previous
import functools

import jax
import jax.numpy as jnp
from jax import lax
from jax.experimental import pallas as pl
from jax.experimental.pallas import tpu as pltpu
from jax.experimental.pallas import tpu_sc as plsc

NUM_FACTORS = 32
VOCAB = 1000001
BATCH = 16384
H0, H1, H2 = 64, 32, 16

NC, NS = 2, 16
NW = NC * NS
BPW = BATCH // NW
CH = 128
NCH = BPW // CH

CBLK = 16384
QB = CBLK // 4
NBLK = pl.cdiv(VOCAB, CBLK)
NPACK = NBLK * QB
BLK = 2048


def _repack_body(in_ref, out_ref):
    x = in_ref[...]
    y = jnp.concatenate([x[:, c * QB:(c + 1) * QB] for c in range(4)], axis=0)
    out_ref[...] = y.T


def _tc_repack(tbl_t):
    return pl.pallas_call(
        _repack_body,
        grid=(NBLK,),
        in_specs=[pl.BlockSpec((NUM_FACTORS, CBLK), lambda i: (0, i))],
        out_specs=pl.BlockSpec((QB, 4 * NUM_FACTORS), lambda i: (i, 0)),
        out_shape=jax.ShapeDtypeStruct((NPACK, 4 * NUM_FACTORS), jnp.float32),
    )(tbl_t)


def _sc_gather(gu, gi, Pp, Qp, Up, Vp):
    mesh = plsc.VectorSubcoreMesh(core_axis_name="c", subcore_axis_name="s")
    out_t = tuple(jax.ShapeDtypeStruct((BATCH, 4 * NUM_FACTORS), jnp.float32)
                  for _ in range(4))

    @functools.partial(
        pl.kernel, mesh=mesh, out_type=out_t,
        scratch_types=[
            pltpu.VMEM((CH,), jnp.int32),
            pltpu.VMEM((CH,), jnp.int32),
            pltpu.VMEM((CH, 4 * NUM_FACTORS), jnp.float32),
            pltpu.VMEM((CH, 4 * NUM_FACTORS), jnp.float32),
            pltpu.VMEM((CH, 4 * NUM_FACTORS), jnp.float32),
            pltpu.VMEM((CH, 4 * NUM_FACTORS), jnp.float32),
            pltpu.SemaphoreType.DMA,
        ],
    )
    def gather_kernel(gu_hbm, gi_hbm, p_hbm, q_hbm, u_hbm, v_hbm,
                      p_out, q_out, u_out, v_out,
                      uidx, iidx, pr, qr, ur, vr, sem):
        wid = lax.axis_index("s") * NC + lax.axis_index("c")
        base = wid * BPW

        def chunk(j):
            off = base + j * CH
            pltpu.sync_copy(gu_hbm.at[pl.ds(off, CH)], uidx)
            pltpu.sync_copy(gi_hbm.at[pl.ds(off, CH)], iidx)
            copies = [
                pltpu.async_copy(p_hbm.at[uidx], pr, sem),
                pltpu.async_copy(u_hbm.at[uidx], ur, sem),
                pltpu.async_copy(q_hbm.at[iidx], qr, sem),
                pltpu.async_copy(v_hbm.at[iidx], vr, sem),
            ]
            for c in copies:
                c.wait()
            sl = pl.ds(off, CH)
            pltpu.sync_copy(pr, p_out.at[sl])
            pltpu.sync_copy(qr, q_out.at[sl])
            pltpu.sync_copy(ur, u_out.at[sl])
            pltpu.sync_copy(vr, v_out.at[sl])

        for j in range(NCH):
            chunk(j)

    return gather_kernel(gu, gi, Pp, Qp, Up, Vp)


def _extract(g, off_ref):
    out = jnp.zeros((g.shape[0], NUM_FACTORS), jnp.float32)
    for c in range(4):
        sel = off_ref == (c * NUM_FACTORS)
        out = jnp.where(sel, g[:, c * NUM_FACTORS:(c + 1) * NUM_FACTORS], out)
    return out


def _mlp_body(p_ref, q_ref, u_ref, v_ref, ou_ref, oi_ref,
              w0_ref, b0_ref, w1_ref, b1_ref, w2_ref, b2_ref, wp_ref,
              out_ref):
    hi = lax.Precision.HIGHEST
    ou = ou_ref[...]
    oi = oi_ref[...]
    p = _extract(p_ref[...], ou)
    q = _extract(q_ref[...], oi)
    u = _extract(u_ref[...], ou)
    v = _extract(v_ref[...], oi)
    gmf = p * q
    w0 = w0_ref[...]
    h = (jnp.dot(u, w0[:NUM_FACTORS], precision=hi)
         + jnp.dot(v, w0[NUM_FACTORS:], precision=hi) + b0_ref[...])
    h = jnp.maximum(h, 0.0)
    h = jnp.maximum(jnp.dot(h, w1_ref[...], precision=hi) + b1_ref[...], 0.0)
    h = jnp.maximum(jnp.dot(h, w2_ref[...], precision=hi) + b2_ref[...], 0.0)
    wp = wp_ref[...]
    out_ref[...] = (jnp.dot(gmf, wp[:NUM_FACTORS], precision=hi)
                    + jnp.dot(h, wp[NUM_FACTORS:], precision=hi))


def _mlp(gp, gq, gu, gv, ou, oi, W0, b0, W1, b1, W2, b2, Wp):
    n_blk = BATCH // BLK
    row_spec = lambda d: pl.BlockSpec((BLK, d), lambda i: (i, 0))
    full = lambda s: pl.BlockSpec(s, lambda i: (0, 0))
    return pl.pallas_call(
        _mlp_body,
        grid=(n_blk,),
        in_specs=[
            row_spec(4 * NUM_FACTORS), row_spec(4 * NUM_FACTORS),
            row_spec(4 * NUM_FACTORS), row_spec(4 * NUM_FACTORS),
            row_spec(1), row_spec(1),
            full((2 * NUM_FACTORS, H0)), full((1, H0)),
            full((H0, H1)), full((1, H1)),
            full((H1, H2)), full((1, H2)),
            full((H2 + NUM_FACTORS, 1)),
        ],
        out_specs=pl.BlockSpec((BLK, 1), lambda i: (i, 0)),
        out_shape=jax.ShapeDtypeStruct((BATCH, 1), jnp.float32),
    )(gp, gq, gu, gv, ou, oi, W0, b0.reshape(1, H0), W1, b1.reshape(1, H1),
      W2, b2.reshape(1, H2), Wp)


def kernel(user_id, item_id, P, Q, U, V, W0, b0, W1, b1, W2, b2, Wp):
    uid = user_id.astype(jnp.int32)
    iid = item_id.astype(jnp.int32)
    last_full = (NBLK - 1) * CBLK
    clamp_start = VOCAB - CBLK

    def packed_coords(idx):
        tail = idx >= last_full
        b = jnp.where(tail, NBLK - 1, idx // CBLK)
        r = jnp.where(tail, idx - clamp_start, idx % CBLK)
        g = b * QB + r % QB
        off = (r // QB) * NUM_FACTORS
        return g, off

    gu, ou = packed_coords(uid)
    gi, oi = packed_coords(iid)
    Pp = _tc_repack(P.T)
    Qp = _tc_repack(Q.T)
    Up = _tc_repack(U.T)
    Vp = _tc_repack(V.T)
    gp, gq, gub, gvb = _sc_gather(gu, gi, Pp, Qp, Up, Vp)
    return _mlp(gp, gq, gub, gvb, ou.reshape(BATCH, 1), oi.reshape(BATCH, 1),
                W0, b0, W1, b1, W2, b2, Wp)


# speedup vs baseline: 6.6724x; 1.1484x over previous
import functools

import jax
import jax.numpy as jnp
from jax import lax
from jax.experimental import pallas as pl
from jax.experimental.pallas import tpu as pltpu
from jax.experimental.pallas import tpu_sc as plsc

NUM_FACTORS = 32
VOCAB = 1000001
BATCH = 16384
H0, H1, H2 = 64, 32, 16

NC, NS = 2, 16
NW = NC * NS
BPW = BATCH // NW
CH = 128
NCH = BPW // CH

CBLK = 16384
QB = CBLK // 4
NBLK = pl.cdiv(VOCAB, CBLK)
NPACK = NBLK * QB
BLK = 2048


def _repack_body(*refs):
    in_refs, out_refs = refs[:4], refs[4:]
    for in_ref, out_ref in zip(in_refs, out_refs):
        x = in_ref[...]
        y = jnp.concatenate(
            [x[:, c * QB:(c + 1) * QB] for c in range(4)], axis=0)
        out_ref[...] = y.T


def _tc_repack4(t0, t1, t2, t3):
    return pl.pallas_call(
        _repack_body,
        grid=(NBLK,),
        in_specs=[pl.BlockSpec((NUM_FACTORS, CBLK), lambda i: (0, i))] * 4,
        out_specs=[pl.BlockSpec((QB, 4 * NUM_FACTORS), lambda i: (i, 0))] * 4,
        out_shape=[jax.ShapeDtypeStruct((NPACK, 4 * NUM_FACTORS),
                                        jnp.float32)] * 4,
    )(t0, t1, t2, t3)


def _sc_gather(gu, gi, Pp, Qp, Up, Vp):
    mesh = plsc.VectorSubcoreMesh(core_axis_name="c", subcore_axis_name="s")
    out_t = tuple(jax.ShapeDtypeStruct((BATCH, 4 * NUM_FACTORS), jnp.float32)
                  for _ in range(4))

    @functools.partial(
        pl.kernel, mesh=mesh, out_type=out_t,
        scratch_types=[
            pltpu.VMEM((CH,), jnp.int32),
            pltpu.VMEM((CH,), jnp.int32),
            pltpu.VMEM((CH, 4 * NUM_FACTORS), jnp.float32),
            pltpu.VMEM((CH, 4 * NUM_FACTORS), jnp.float32),
            pltpu.VMEM((CH, 4 * NUM_FACTORS), jnp.float32),
            pltpu.VMEM((CH, 4 * NUM_FACTORS), jnp.float32),
            pltpu.SemaphoreType.DMA,
        ],
    )
    def gather_kernel(gu_hbm, gi_hbm, p_hbm, q_hbm, u_hbm, v_hbm,
                      p_out, q_out, u_out, v_out,
                      uidx, iidx, pr, qr, ur, vr, sem):
        wid = lax.axis_index("s") * NC + lax.axis_index("c")
        base = wid * BPW

        def chunk(j):
            off = base + j * CH
            pltpu.sync_copy(gu_hbm.at[pl.ds(off, CH)], uidx)
            pltpu.sync_copy(gi_hbm.at[pl.ds(off, CH)], iidx)
            copies = [
                pltpu.async_copy(p_hbm.at[uidx], pr, sem),
                pltpu.async_copy(u_hbm.at[uidx], ur, sem),
                pltpu.async_copy(q_hbm.at[iidx], qr, sem),
                pltpu.async_copy(v_hbm.at[iidx], vr, sem),
            ]
            for c in copies:
                c.wait()
            sl = pl.ds(off, CH)
            pltpu.sync_copy(pr, p_out.at[sl])
            pltpu.sync_copy(qr, q_out.at[sl])
            pltpu.sync_copy(ur, u_out.at[sl])
            pltpu.sync_copy(vr, v_out.at[sl])

        for j in range(NCH):
            chunk(j)

    return gather_kernel(gu, gi, Pp, Qp, Up, Vp)


def _extract(g, off_ref):
    out = jnp.zeros((g.shape[0], NUM_FACTORS), jnp.float32)
    for c in range(4):
        sel = off_ref == (c * NUM_FACTORS)
        out = jnp.where(sel, g[:, c * NUM_FACTORS:(c + 1) * NUM_FACTORS], out)
    return out


def _mlp_body(p_ref, q_ref, u_ref, v_ref, ou_ref, oi_ref,
              w0_ref, b0_ref, w1_ref, b1_ref, w2_ref, b2_ref, wp_ref,
              out_ref):
    hi = lax.Precision.HIGHEST
    ou = ou_ref[...]
    oi = oi_ref[...]
    p = _extract(p_ref[...], ou)
    q = _extract(q_ref[...], oi)
    u = _extract(u_ref[...], ou)
    v = _extract(v_ref[...], oi)
    gmf = p * q
    w0 = w0_ref[...]
    h = (jnp.dot(u, w0[:NUM_FACTORS], precision=hi)
         + jnp.dot(v, w0[NUM_FACTORS:], precision=hi) + b0_ref[...])
    h = jnp.maximum(h, 0.0)
    h = jnp.maximum(jnp.dot(h, w1_ref[...], precision=hi) + b1_ref[...], 0.0)
    h = jnp.maximum(jnp.dot(h, w2_ref[...], precision=hi) + b2_ref[...], 0.0)
    wp = wp_ref[...]
    out_ref[...] = (jnp.dot(gmf, wp[:NUM_FACTORS], precision=hi)
                    + jnp.dot(h, wp[NUM_FACTORS:], precision=hi))


def _mlp(gp, gq, gu, gv, ou, oi, W0, b0, W1, b1, W2, b2, Wp):
    n_blk = BATCH // BLK
    row_spec = lambda d: pl.BlockSpec((BLK, d), lambda i: (i, 0))
    full = lambda s: pl.BlockSpec(s, lambda i: (0, 0))
    return pl.pallas_call(
        _mlp_body,
        grid=(n_blk,),
        in_specs=[
            row_spec(4 * NUM_FACTORS), row_spec(4 * NUM_FACTORS),
            row_spec(4 * NUM_FACTORS), row_spec(4 * NUM_FACTORS),
            row_spec(1), row_spec(1),
            full((2 * NUM_FACTORS, H0)), full((1, H0)),
            full((H0, H1)), full((1, H1)),
            full((H1, H2)), full((1, H2)),
            full((H2 + NUM_FACTORS, 1)),
        ],
        out_specs=pl.BlockSpec((BLK, 1), lambda i: (i, 0)),
        out_shape=jax.ShapeDtypeStruct((BATCH, 1), jnp.float32),
    )(gp, gq, gu, gv, ou, oi, W0, b0.reshape(1, H0), W1, b1.reshape(1, H1),
      W2, b2.reshape(1, H2), Wp)


def kernel(user_id, item_id, P, Q, U, V, W0, b0, W1, b1, W2, b2, Wp):
    uid = user_id.astype(jnp.int32)
    iid = item_id.astype(jnp.int32)
    last_full = (NBLK - 1) * CBLK
    clamp_start = VOCAB - CBLK

    def packed_coords(idx):
        tail = idx >= last_full
        b = jnp.where(tail, NBLK - 1, idx // CBLK)
        r = jnp.where(tail, idx - clamp_start, idx % CBLK)
        g = b * QB + r % QB
        off = (r // QB) * NUM_FACTORS
        return g, off

    gu, ou = packed_coords(uid)
    gi, oi = packed_coords(iid)
    Pp, Qp, Up, Vp = _tc_repack4(P.T, Q.T, U.T, V.T)
    gp, gq, gub, gvb = _sc_gather(gu, gi, Pp, Qp, Up, Vp)
    return _mlp(gp, gq, gub, gvb, ou.reshape(BATCH, 1), oi.reshape(BATCH, 1),
                W0, b0, W1, b1, W2, b2, Wp)
